# baseline (speedup 1.0000x reference)
import jax
import jax.numpy as jnp
from jax import lax
from jax.experimental import pallas as pl
from jax.experimental.pallas import tpu as pltpu
from jax.experimental.pallas import tpu_sc as plsc

B = 128
C = 32767
DEPTH = 15
NC = 2
NS = 16
LANES = 16
NW = 32
SPW = B // NW
PREF = 2048
NDEEP = DEPTH - 1 - 10


def _softplus_sc(x):
    u = jnp.exp(-jnp.abs(x))
    v = 1.0 + u
    y = u * (1.0 - u * (0.5 - u * 0.25))
    y = y + v * jnp.exp(-y) - 1.0
    y = y + v * jnp.exp(-y) - 1.0
    return jnp.maximum(x, 0.0) + y


def _decode_body(o_hbm, t_hbm, e_hbm,
                 obuf, tbuf, deep_o, deep_t, ebuf, sem):
    wid = lax.axis_index("s") * NC + lax.axis_index("c")
    lane = lax.iota(jnp.int32, LANES)
    row0 = pl.multiple_of((wid // 2) * 8, 8)
    samp = jnp.where(lane < SPW, lane, 0)
    r = (wid % 2) * 4 + samp

    cp0 = pltpu.async_copy(o_hbm.at[pl.ds(row0, 8), pl.ds(0, PREF)], obuf, sem)
    cp1 = pltpu.async_copy(t_hbm.at[pl.ds(row0, 8), pl.ds(0, PREF)], tbuf, sem)
    cp0.wait()
    cp1.wait()

    zf = jnp.zeros((LANES,), jnp.float32)

    l0 = plsc.load_gather(obuf, [r, jnp.zeros((LANES,), jnp.int32)])
    state0 = (jnp.zeros((LANES,), jnp.int32),
              jnp.where(l0 > 0.0, 1.0, 0.0), zf, zf)

    def step(st, gathered):
        cur, active, run, extra = st
        c1 = 2 * cur + 1
        c2 = c1 + 1
        l1, t1, l2, t2 = gathered
        take2 = l2 > l1
        lsel = jnp.where(take2, l2, l1)
        tsel = jnp.where(take2, t2, t1)
        cur = jnp.where(take2, c2, c1)
        active = jnp.where(lsel > 0.0, active, 0.0)
        m = (active > 0.0) & (tsel == 0.0)
        run = jnp.where(m, run + 1.0, 0.0)
        w = jnp.where(lane < SPW, run, 0.0)
        extra = extra + w * _softplus_sc(lsel)
        return (cur, active, run, extra)

    def pref_body(lvl, st):
        c1 = 2 * st[0] + 1
        c2 = c1 + 1
        return step(st, (plsc.load_gather(obuf, [r, c1]),
                         plsc.load_gather(tbuf, [r, c1]),
                         plsc.load_gather(obuf, [r, c2]),
                         plsc.load_gather(tbuf, [r, c2])))

    state = lax.fori_loop(1, 11, pref_body, state0)

    def wbase(c, j):
        s = (2 ** j) * c + (2 ** j - 1)
        base = jnp.right_shift(s, 7) * 128
        if j == NDEEP:
            base = jnp.minimum(base, 32512)
        return base

    cur10 = state[0]
    cps = []
    for k in range(SPW):
        ck = cur10[k]
        for j in range(1, NDEEP + 1):
            base = pl.multiple_of(wbase(ck, j), 128)
            slot = k * NDEEP + (j - 1)
            src = pl.ds(base, 256)
            cps.append(pltpu.async_copy(
                o_hbm.at[pl.ds(row0, 8), src], deep_o.at[slot], sem))
            cps.append(pltpu.async_copy(
                t_hbm.at[pl.ds(row0, 8), src], deep_t.at[slot], sem))
    for cp in cps:
        cp.wait()

    def deep_body(lvl, st):
        j = lvl - 10
        s_vec = jnp.left_shift(cur10, j) + (jnp.left_shift(1, j) - 1)
        base_vec = jnp.right_shift(s_vec, 7) * 128
        base_vec = jnp.where(j == NDEEP, jnp.minimum(base_vec, 32512), base_vec)
        slot_vec = samp * NDEEP + (j - 1)
        c1 = 2 * st[0] + 1
        c2 = c1 + 1
        r1 = c1 - base_vec
        r2 = c2 - base_vec
        return step(st, (plsc.load_gather(deep_o, [slot_vec, r, r1]),
                         plsc.load_gather(deep_t, [slot_vec, r, r1]),
                         plsc.load_gather(deep_o, [slot_vec, r, r2]),
                         plsc.load_gather(deep_t, [slot_vec, r, r2])))

    state = lax.fori_loop(11, DEPTH, deep_body, state)

    for k in range(8):
        ebuf[k] = state[3] if k == 0 else zf
    pltpu.sync_copy(ebuf, e_hbm.at[wid])


def _make_decode():
    return pl.kernel(
        _decode_body,
        out_type=jax.ShapeDtypeStruct((NW, 8, LANES), jnp.float32),
        mesh=plsc.VectorSubcoreMesh(core_axis_name="c", subcore_axis_name="s",
                                    num_cores=NC, num_subcores=NS),
        scratch_types=[
            pltpu.VMEM((8, PREF), jnp.float32),
            pltpu.VMEM((8, PREF), jnp.float32),
            pltpu.VMEM((SPW * NDEEP, 8, 256), jnp.float32),
            pltpu.VMEM((SPW * NDEEP, 8, 256), jnp.float32),
            pltpu.VMEM((8, LANES), jnp.float32),
            pltpu.SemaphoreType.DMA,
        ],
        compiler_params=pltpu.CompilerParams(needs_layout_passes=False),
    )


BLK = 4096
NBLK = 8


def _sum_body(o_ref, t_ref, out_ref, acc_ref):
    i = pl.program_id(0)
    col = i * BLK + lax.broadcasted_iota(jnp.int32, (8, BLK), 1)
    mask = col < C

    @pl.when(i == 0)
    def _():
        acc_ref[...] = jnp.zeros((8, BLK), jnp.float32)

    acc = acc_ref[...]
    for rr in range(0, B, 8):
        xb = o_ref[pl.ds(rr, 8), :]
        tb = t_ref[pl.ds(rr, 8), :]
        u = jnp.exp2(jnp.abs(xb) * (-1.4426950408889634))
        sp = jnp.maximum(xb, 0.0) + 0.6931471805599453 * jnp.log2(1.0 + u)
        acc = acc + jnp.where(mask, sp - xb * tb, 0.0)
    acc_ref[...] = acc

    @pl.when(i == NBLK - 1)
    def _():
        out_ref[0, 0] = jnp.sum(acc)


_sum = pl.pallas_call(
    _sum_body,
    grid=(NBLK,),
    in_specs=[
        pl.BlockSpec((B, BLK), lambda i: (0, i)),
        pl.BlockSpec((B, BLK), lambda i: (0, i)),
    ],
    out_specs=pl.BlockSpec(memory_space=pltpu.SMEM),
    out_shape=jax.ShapeDtypeStruct((1, 1), jnp.float32),
    scratch_shapes=[pltpu.VMEM((8, BLK), jnp.float32)],
)


@jax.jit
def kernel(outputs, targets):
    extra = _make_decode()(outputs, targets)
    dense = _sum(outputs, targets)
    return (dense[0, 0] + jnp.sum(extra)) / (B * C)


# speedup vs baseline: 1.0073x; 1.0073x over previous
import jax
import jax.numpy as jnp
from jax import lax
from jax.experimental import pallas as pl
from jax.experimental.pallas import tpu as pltpu
from jax.experimental.pallas import tpu_sc as plsc

B = 128
C = 32767
DEPTH = 15
NC = 2
NS = 16
LANES = 16
NW = 32
SPW = B // NW
PREF = 2048
NDEEP = DEPTH - 1 - 10


def _softplus_sc(x):
    u = jnp.exp(-jnp.abs(x))
    v = 1.0 + u
    y = u * (1.0 - u * (0.5 - u * 0.25))
    y = y + v * jnp.exp(-y) - 1.0
    y = y + v * jnp.exp(-y) - 1.0
    return jnp.maximum(x, 0.0) + y


def _decode_body(o_hbm, t_hbm, e_hbm,
                 obuf, tbuf, deep_o, deep_t, ebuf, sem):
    wid = lax.axis_index("s") * NC + lax.axis_index("c")
    lane = lax.iota(jnp.int32, LANES)
    row0 = pl.multiple_of((wid // 2) * 8, 8)
    samp = jnp.where(lane < SPW, lane, 0)
    r = (wid % 2) * 4 + samp

    cp0 = pltpu.async_copy(o_hbm.at[pl.ds(row0, 8), pl.ds(0, PREF)], obuf, sem)
    cp1 = pltpu.async_copy(t_hbm.at[pl.ds(row0, 8), pl.ds(0, PREF)], tbuf, sem)
    cp0.wait()
    cp1.wait()

    zf = jnp.zeros((LANES,), jnp.float32)

    l0 = plsc.load_gather(obuf, [r, jnp.zeros((LANES,), jnp.int32)])
    state0 = (jnp.zeros((LANES,), jnp.int32),
              jnp.where(l0 > 0.0, 1.0, 0.0), zf, zf)

    def step(st, gathered):
        cur, active, run, extra = st
        c1 = 2 * cur + 1
        c2 = c1 + 1
        l1, t1, l2, t2 = gathered
        take2 = l2 > l1
        lsel = jnp.where(take2, l2, l1)
        tsel = jnp.where(take2, t2, t1)
        cur = jnp.where(take2, c2, c1)
        active = jnp.where(lsel > 0.0, active, 0.0)
        m = (active > 0.0) & (tsel == 0.0)
        run = jnp.where(m, run + 1.0, 0.0)
        w = jnp.where(lane < SPW, run, 0.0)
        extra = extra + w * _softplus_sc(lsel)
        return (cur, active, run, extra)

    def pref_body(lvl, st):
        c1 = 2 * st[0] + 1
        c2 = c1 + 1
        return step(st, (plsc.load_gather(obuf, [r, c1]),
                         plsc.load_gather(tbuf, [r, c1]),
                         plsc.load_gather(obuf, [r, c2]),
                         plsc.load_gather(tbuf, [r, c2])))

    state = state0
    for lvl in range(1, 11):
        state = pref_body(lvl, state)

    def wbase(c, j):
        s = (2 ** j) * c + (2 ** j - 1)
        base = jnp.right_shift(s, 7) * 128
        if j == NDEEP:
            base = jnp.minimum(base, 32512)
        return base

    cur10 = state[0]
    cps = []
    for k in range(SPW):
        ck = cur10[k]
        for j in range(1, NDEEP + 1):
            base = pl.multiple_of(wbase(ck, j), 128)
            slot = k * NDEEP + (j - 1)
            src = pl.ds(base, 256)
            cps.append(pltpu.async_copy(
                o_hbm.at[pl.ds(row0, 8), src], deep_o.at[slot], sem))
            cps.append(pltpu.async_copy(
                t_hbm.at[pl.ds(row0, 8), src], deep_t.at[slot], sem))
    for cp in cps:
        cp.wait()

    def deep_body(lvl, st):
        j = lvl - 10
        s_vec = jnp.left_shift(cur10, j) + (jnp.left_shift(1, j) - 1)
        base_vec = jnp.right_shift(s_vec, 7) * 128
        base_vec = jnp.where(j == NDEEP, jnp.minimum(base_vec, 32512), base_vec)
        slot_vec = samp * NDEEP + (j - 1)
        c1 = 2 * st[0] + 1
        c2 = c1 + 1
        r1 = c1 - base_vec
        r2 = c2 - base_vec
        return step(st, (plsc.load_gather(deep_o, [slot_vec, r, r1]),
                         plsc.load_gather(deep_t, [slot_vec, r, r1]),
                         plsc.load_gather(deep_o, [slot_vec, r, r2]),
                         plsc.load_gather(deep_t, [slot_vec, r, r2])))

    for lvl in range(11, DEPTH):
        state = deep_body(lvl, state)

    for k in range(8):
        ebuf[k] = state[3] if k == 0 else zf
    pltpu.sync_copy(ebuf, e_hbm.at[wid])


def _make_decode():
    return pl.kernel(
        _decode_body,
        out_type=jax.ShapeDtypeStruct((NW, 8, LANES), jnp.float32),
        mesh=plsc.VectorSubcoreMesh(core_axis_name="c", subcore_axis_name="s",
                                    num_cores=NC, num_subcores=NS),
        scratch_types=[
            pltpu.VMEM((8, PREF), jnp.float32),
            pltpu.VMEM((8, PREF), jnp.float32),
            pltpu.VMEM((SPW * NDEEP, 8, 256), jnp.float32),
            pltpu.VMEM((SPW * NDEEP, 8, 256), jnp.float32),
            pltpu.VMEM((8, LANES), jnp.float32),
            pltpu.SemaphoreType.DMA,
        ],
        compiler_params=pltpu.CompilerParams(needs_layout_passes=False),
    )


BLK = 4096
NBLK = 8


def _sum_body(o_ref, t_ref, out_ref, acc_ref):
    i = pl.program_id(0)
    col = i * BLK + lax.broadcasted_iota(jnp.int32, (8, BLK), 1)
    mask = col < C

    @pl.when(i == 0)
    def _():
        acc_ref[...] = jnp.zeros((8, BLK), jnp.float32)

    acc = acc_ref[...]
    for rr in range(0, B, 8):
        xb = o_ref[pl.ds(rr, 8), :]
        tb = t_ref[pl.ds(rr, 8), :]
        u = jnp.exp2(jnp.abs(xb) * (-1.4426950408889634))
        sp = jnp.maximum(xb, 0.0) + 0.6931471805599453 * jnp.log2(1.0 + u)
        acc = acc + jnp.where(mask, sp - xb * tb, 0.0)
    acc_ref[...] = acc

    @pl.when(i == NBLK - 1)
    def _():
        out_ref[0, 0] = jnp.sum(acc)


_sum = pl.pallas_call(
    _sum_body,
    grid=(NBLK,),
    in_specs=[
        pl.BlockSpec((B, BLK), lambda i: (0, i)),
        pl.BlockSpec((B, BLK), lambda i: (0, i)),
    ],
    out_specs=pl.BlockSpec(memory_space=pltpu.SMEM),
    out_shape=jax.ShapeDtypeStruct((1, 1), jnp.float32),
    scratch_shapes=[pltpu.VMEM((8, BLK), jnp.float32)],
)


@jax.jit
def kernel(outputs, targets):
    extra = _make_decode()(outputs, targets)
    dense = _sum(outputs, targets)
    return (dense[0, 0] + jnp.sum(extra)) / (B * C)
